# baseline (speedup 1.0000x reference)
import jax
import jax.numpy as jnp
from jax import lax
from jax.experimental import pallas as pl
from jax.experimental.pallas import tpu as pltpu
from jax.experimental.pallas import tpu_sc as plsc

N = 10000
E = 320000
D = 512
L = 16
NC = 2
NS = 16
NT = NC * NS
WN = 320
DH = D // 2
SCAN = 3200
NCHUNK = E // SCAN
CD_SZ = SCAN + 64
LASTW = N - (NT - 1) * WN
BLK = 1000


def _edge_body(dst_hbm, src_hbm, kh_hbm, qvh_hbm, agg_hbm,
               eb_d0, eb_s0, eb_d1, eb_s1, cd_v, cs_v,
               kd0, qv0, kd1, qv1, acc,
               sem0, sem1, semA, semB):
    cid = lax.axis_index("c")
    sid = lax.axis_index("s")
    t = cid * NS + sid
    wlo = t * WN
    lane = lax.iota(jnp.int32, L)

    def issue_gather(b, n, h, kdb, qvb, sem):
        base = b * L
        d = cd_v[pl.ds(base, L)]
        s = cs_v[pl.ds(base, L)]
        valid = (lane + base) < n
        gd = jnp.where(valid, d + h * N, 0)
        gs = jnp.where(valid, s + h * N, 0)
        pltpu.async_copy(kh_hbm.at[gd], kdb, sem)
        pltpu.async_copy(qvh_hbm.at[gs], qvb, sem)

    def wait_pair(kdb, qvb, sem):
        pltpu.make_async_copy(kh_hbm.at[pl.ds(0, L)], kdb, sem).wait()
        pltpu.make_async_copy(qvh_hbm.at[pl.ds(0, L)], qvb, sem).wait()

    def consume(b, n, kdb, qvb):
        base = b * L
        d = cd_v[pl.ds(base, L)]
        valid = (lane + base) < n
        lrow = jnp.where(valid, d - wlo, 0)

        def col(c4, _):
            for u in range(4):
                cvec = ((c4 * 4 + u) + lane) & (DH - 1)
                kcol = plsc.load_gather(kdb, [lane, cvec])
                qcol = plsc.load_gather(qvb, [lane, cvec])
                vcol = plsc.load_gather(qvb, [lane, cvec + DH])
                y = 1.0 + jnp.minimum(jnp.exp(-(kcol + qcol)), jnp.float32(1e30))
                r = plsc.bitcast(jnp.int32(0x7EF127EA) - plsc.bitcast(y, jnp.int32), jnp.float32)
                r = r * (2.0 - y * r)
                r = r * (2.0 - y * r)
                r = r * (2.0 - y * r)
                g = vcol * r
                plsc.addupdate_scatter(acc, [lrow, cvec], g, mask=valid)
            return 0
        lax.fori_loop(0, DH // 4, col, 0)

    def issue_ids(c2, buf_d, buf_s, semc):
        off = c2 * SCAN
        pltpu.async_copy(dst_hbm.at[pl.ds(off, SCAN)], buf_d, semc)
        pltpu.async_copy(src_hbm.at[pl.ds(off, SCAN)], buf_s, semc)

    for h in range(2):
        def zr(r, _):
            for c in range(DH // L):
                acc[r, pl.ds(c * L, L)] = jnp.zeros((L,), jnp.float32)
            return 0
        lax.fori_loop(0, WN, zr, 0)

        def do_chunk(buf_d, buf_s, semc):
            pltpu.make_async_copy(dst_hbm.at[pl.ds(0, SCAN)], buf_d, semc).wait()
            pltpu.make_async_copy(src_hbm.at[pl.ds(0, SCAN)], buf_s, semc).wait()

            def scan_body(it, n):
                for u in range(2):
                    d = buf_d[pl.ds((it * 2 + u) * L, L)]
                    s = buf_s[pl.ds((it * 2 + u) * L, L)]
                    m = plsc.bitcast(d - wlo, jnp.uint32) < jnp.uint32(WN)
                    mi = jnp.where(m, jnp.int32(1), jnp.int32(0))
                    pos = n + plsc.cumsum(mi) - 1
                    plsc.store_scatter(cd_v, [pos], d, mask=m)
                    plsc.store_scatter(cs_v, [pos], s, mask=m)
                    n = n + jnp.sum(mi)
                return n
            n = lax.fori_loop(0, SCAN // L // 2, scan_body, jnp.int32(0))

            nb = (n + L - 1) // L

            @pl.when(nb > 0)
            def _():
                issue_gather(jnp.int32(0), n, h, kd0, qv0, sem0)

            def pipe(i, _):
                b0 = 2 * i

                @pl.when(b0 + 1 < nb)
                def _():
                    issue_gather(b0 + 1, n, h, kd1, qv1, sem1)
                wait_pair(kd0, qv0, sem0)
                consume(b0, n, kd0, qv0)

                @pl.when(b0 + 2 < nb)
                def _():
                    issue_gather(b0 + 2, n, h, kd0, qv0, sem0)

                @pl.when(b0 + 1 < nb)
                def _():
                    wait_pair(kd1, qv1, sem1)
                    consume(b0 + 1, n, kd1, qv1)
                return 0
            lax.fori_loop(0, (nb + 1) // 2, pipe, 0)

        issue_ids(jnp.int32(0), eb_d0, eb_s0, semA)

        def chunk_pair(i, _):
            issue_ids(2 * i + 1, eb_d1, eb_s1, semB)
            do_chunk(eb_d0, eb_s0, semA)

            @pl.when(2 * i + 2 < NCHUNK)
            def _():
                issue_ids(2 * i + 2, eb_d0, eb_s0, semA)
            do_chunk(eb_d1, eb_s1, semB)
            return 0
        lax.fori_loop(0, NCHUNK // 2, chunk_pair, 0)

        @pl.when(t < NT - 1)
        def _():
            pltpu.sync_copy(acc, agg_hbm.at[pl.ds(wlo, WN), pl.ds(h * DH, DH)])

        @pl.when(t == NT - 1)
        def _():
            pltpu.sync_copy(acc.at[pl.ds(0, LASTW)],
                            agg_hbm.at[pl.ds(wlo, LASTW), pl.ds(h * DH, DH)])


def _edge_agg(dst, src, kh, qvh):
    mesh = plsc.VectorSubcoreMesh(core_axis_name="c", subcore_axis_name="s",
                                  num_cores=NC, num_subcores=NS)
    f = pl.kernel(
        _edge_body,
        out_type=jax.ShapeDtypeStruct((N, D), jnp.float32),
        mesh=mesh,
        compiler_params=pltpu.CompilerParams(needs_layout_passes=False),
        scratch_types=[
            pltpu.VMEM((SCAN,), jnp.int32),
            pltpu.VMEM((SCAN,), jnp.int32),
            pltpu.VMEM((SCAN,), jnp.int32),
            pltpu.VMEM((SCAN,), jnp.int32),
            pltpu.VMEM((CD_SZ,), jnp.int32),
            pltpu.VMEM((CD_SZ,), jnp.int32),
            pltpu.VMEM((L, DH), jnp.float32),
            pltpu.VMEM((L, D), jnp.float32),
            pltpu.VMEM((L, DH), jnp.float32),
            pltpu.VMEM((L, D), jnp.float32),
            pltpu.VMEM((WN, DH), jnp.float32),
            pltpu.SemaphoreType.DMA,
            pltpu.SemaphoreType.DMA,
            pltpu.SemaphoreType.DMA,
            pltpu.SemaphoreType.DMA,
        ],
    )
    return f(dst, src, kh, qvh)


def _linstats_body(x_ref, W_ref, b_ref, o_ref, st_ref):
    i = pl.program_id(0)
    o = jnp.dot(x_ref[...], W_ref[...], preferred_element_type=jnp.float32) + b_ref[...]
    o_ref[...] = o

    @pl.when(i == 0)
    def _():
        st_ref[...] = jnp.zeros_like(st_ref)
    s1 = jnp.sum(o, axis=0)[None, :]
    s2 = jnp.sum(o * o, axis=0)[None, :]
    st_ref[...] += jnp.concatenate([s1, s2], axis=0)


def _linstats(x, W, b):
    K = x.shape[1]
    return pl.pallas_call(
        _linstats_body,
        grid=(N // BLK,),
        in_specs=[
            pl.BlockSpec((BLK, K), lambda i: (i, 0)),
            pl.BlockSpec((K, D), lambda i: (0, 0)),
            pl.BlockSpec((1, D), lambda i: (0, 0)),
        ],
        out_specs=[
            pl.BlockSpec((BLK, D), lambda i: (i, 0)),
            pl.BlockSpec((2, D), lambda i: (0, 0)),
        ],
        out_shape=[
            jax.ShapeDtypeStruct((N, D), jnp.float32),
            jax.ShapeDtypeStruct((2, D), jnp.float32),
        ],
    )(x, W, b)


def _bn_block(o, st_ref, g_ref, bt_ref):
    m = st_ref[0:1, :] / N
    v = st_ref[1:2, :] / N - m * m
    o = g_ref[...] * (o - m) * lax.rsqrt(v + 1e-5) + bt_ref[...]
    o = jnp.maximum(o, 0.0)
    nrm = jnp.sqrt(jnp.sum(o * o, axis=1, keepdims=True))
    return o / jnp.maximum(nrm, 1e-12)


def _bnact_body(o_ref, st_ref, g_ref, bt_ref, h_ref):
    h_ref[...] = _bn_block(o_ref[...], st_ref, g_ref, bt_ref)


def _bnact(o, st, g, bt):
    return pl.pallas_call(
        _bnact_body,
        grid=(N // BLK,),
        in_specs=[
            pl.BlockSpec((BLK, D), lambda i: (i, 0)),
            pl.BlockSpec((2, D), lambda i: (0, 0)),
            pl.BlockSpec((1, D), lambda i: (0, 0)),
            pl.BlockSpec((1, D), lambda i: (0, 0)),
        ],
        out_specs=pl.BlockSpec((BLK, D), lambda i: (i, 0)),
        out_shape=jax.ShapeDtypeStruct((N, D), jnp.float32),
    )(o, st, g, bt)


def _mm_body(h_ref, W1_ref, W2_ref, W34_ref, hw1_ref, kh_ref, qvh_ref):
    h = h_ref[...]
    hw1_ref[...] = jnp.dot(h, W1_ref[...], preferred_element_type=jnp.float32)
    k = jnp.dot(h, W2_ref[...], preferred_element_type=jnp.float32)
    kh_ref[0] = k[:, :DH]
    kh_ref[1] = k[:, DH:]
    qv = jnp.dot(h, W34_ref[...], preferred_element_type=jnp.float32)
    qvh_ref[0] = jnp.concatenate([qv[:, :DH], qv[:, D:D + DH]], axis=1)
    qvh_ref[1] = jnp.concatenate([qv[:, DH:D], qv[:, D + DH:]], axis=1)


def _mm(h, W1, W2, W34):
    return pl.pallas_call(
        _mm_body,
        grid=(N // BLK,),
        in_specs=[
            pl.BlockSpec((BLK, D), lambda i: (i, 0)),
            pl.BlockSpec((D, D), lambda i: (0, 0)),
            pl.BlockSpec((D, D), lambda i: (0, 0)),
            pl.BlockSpec((D, 2 * D), lambda i: (0, 0)),
        ],
        out_specs=[
            pl.BlockSpec((BLK, D), lambda i: (i, 0)),
            pl.BlockSpec((2, BLK, DH), lambda i: (0, i, 0)),
            pl.BlockSpec((2, BLK, D), lambda i: (0, i, 0)),
        ],
        out_shape=[
            jax.ShapeDtypeStruct((N, D), jnp.float32),
            jax.ShapeDtypeStruct((2, N, DH), jnp.float32),
            jax.ShapeDtypeStruct((2, N, D), jnp.float32),
        ],
    )(h, W1, W2, W34)


def _stats_body(hw1_ref, agg_ref, wb_ref, st_ref):
    i = pl.program_id(0)
    o = hw1_ref[...] + agg_ref[...] + wb_ref[...]

    @pl.when(i == 0)
    def _():
        st_ref[...] = jnp.zeros_like(st_ref)
    s1 = jnp.sum(o, axis=0)[None, :]
    s2 = jnp.sum(o * o, axis=0)[None, :]
    st_ref[...] += jnp.concatenate([s1, s2], axis=0)


def _stats(hw1, agg, wb):
    return pl.pallas_call(
        _stats_body,
        grid=(N // BLK,),
        in_specs=[
            pl.BlockSpec((BLK, D), lambda i: (i, 0)),
            pl.BlockSpec((BLK, D), lambda i: (i, 0)),
            pl.BlockSpec((1, D), lambda i: (0, 0)),
        ],
        out_specs=pl.BlockSpec((2, D), lambda i: (0, 0)),
        out_shape=jax.ShapeDtypeStruct((2, D), jnp.float32),
    )(hw1, agg, wb)


def _resupd_body(h_ref, hw1_ref, agg_ref, wb_ref, st_ref, g_ref, bt_ref, hn_ref):
    o = hw1_ref[...] + agg_ref[...] + wb_ref[...]
    o = _bn_block(o, st_ref, g_ref, bt_ref)
    hn_ref[...] = h_ref[...] + o


def _resupd(h, hw1, agg, wb, st, g, bt):
    return pl.pallas_call(
        _resupd_body,
        grid=(N // BLK,),
        in_specs=[
            pl.BlockSpec((BLK, D), lambda i: (i, 0)),
            pl.BlockSpec((BLK, D), lambda i: (i, 0)),
            pl.BlockSpec((BLK, D), lambda i: (i, 0)),
            pl.BlockSpec((1, D), lambda i: (0, 0)),
            pl.BlockSpec((2, D), lambda i: (0, 0)),
            pl.BlockSpec((1, D), lambda i: (0, 0)),
            pl.BlockSpec((1, D), lambda i: (0, 0)),
        ],
        out_specs=pl.BlockSpec((BLK, D), lambda i: (i, 0)),
        out_shape=jax.ShapeDtypeStruct((N, D), jnp.float32),
    )(h, hw1, agg, wb, st, g, bt)


def _post1_body(h_ref, W_ref, b_ref, z1_ref, st_ref):
    i = pl.program_id(0)
    h = h_ref[...]
    nrm = jnp.sqrt(jnp.sum(h * h, axis=1, keepdims=True))
    h = h / jnp.maximum(nrm, 1e-12)
    o = jnp.dot(h, W_ref[...], preferred_element_type=jnp.float32) + b_ref[...]
    z1_ref[...] = o

    @pl.when(i == 0)
    def _():
        st_ref[...] = jnp.zeros_like(st_ref)
    s1 = jnp.sum(o, axis=0)[None, :]
    s2 = jnp.sum(o * o, axis=0)[None, :]
    st_ref[...] += jnp.concatenate([s1, s2], axis=0)


def _post1(h, W, b):
    return pl.pallas_call(
        _post1_body,
        grid=(N // BLK,),
        in_specs=[
            pl.BlockSpec((BLK, D), lambda i: (i, 0)),
            pl.BlockSpec((D, D), lambda i: (0, 0)),
            pl.BlockSpec((1, D), lambda i: (0, 0)),
        ],
        out_specs=[
            pl.BlockSpec((BLK, D), lambda i: (i, 0)),
            pl.BlockSpec((2, D), lambda i: (0, 0)),
        ],
        out_shape=[
            jax.ShapeDtypeStruct((N, D), jnp.float32),
            jax.ShapeDtypeStruct((2, D), jnp.float32),
        ],
    )(h, W, b)


def _post2_body(z1_ref, st_ref, g_ref, bt_ref, W2_ref, b2_ref, z_ref):
    o = z1_ref[...]
    m = st_ref[0:1, :] / N
    v = st_ref[1:2, :] / N - m * m
    o = g_ref[...] * (o - m) * lax.rsqrt(v + 1e-5) + bt_ref[...]
    o = jnp.maximum(o, 0.0)
    z_ref[...] = jnp.dot(o, W2_ref[...], preferred_element_type=jnp.float32) + b2_ref[...]


def _post2(z1, st, g, bt, W2, b2):
    DO = W2.shape[1]
    return pl.pallas_call(
        _post2_body,
        grid=(N // BLK,),
        in_specs=[
            pl.BlockSpec((BLK, D), lambda i: (i, 0)),
            pl.BlockSpec((2, D), lambda i: (0, 0)),
            pl.BlockSpec((1, D), lambda i: (0, 0)),
            pl.BlockSpec((1, D), lambda i: (0, 0)),
            pl.BlockSpec((D, DO), lambda i: (0, 0)),
            pl.BlockSpec((1, DO), lambda i: (0, 0)),
        ],
        out_specs=pl.BlockSpec((BLK, DO), lambda i: (i, 0)),
        out_shape=jax.ShapeDtypeStruct((N, DO), jnp.float32),
    )(z1, st, g, bt, W2, b2)


def kernel(x, edge_index, pre_W, pre_b, pre_g, pre_bt, W1, W2, W3, W4, Wb,
           g_mp, b_mp, h1_W, h1_b, h1_g, h1_bt, h2_W, h2_b):
    src = edge_index[0].astype(jnp.int32)
    dst = edge_index[1].astype(jnp.int32)

    pre, st = _linstats(x, pre_W, pre_b.reshape(1, D))
    h = _bnact(pre, st, pre_g.reshape(1, D), pre_bt.reshape(1, D))

    W34 = jnp.concatenate([W3, W4], axis=-1)
    xs = (W1, W2, W34, Wb[:, None, :], g_mp[:, None, :], b_mp[:, None, :])

    def layer(h, ws):
        W1l, W2l, W34l, wbl, gl, bl = ws
        hw1, kh, qvh = _mm(h, W1l, W2l, W34l)
        agg = _edge_agg(dst, src, kh.reshape(2 * N, DH), qvh.reshape(2 * N, D))
        stl = _stats(hw1, agg, wbl)
        h = _resupd(h, hw1, agg, wbl, stl, gl, bl)
        return h, None

    h, _ = lax.scan(layer, h, xs)

    z1, st = _post1(h, h1_W, h1_b.reshape(1, D))
    z = _post2(z1, st, h1_g.reshape(1, D), h1_bt.reshape(1, D),
               h2_W, h2_b.reshape(1, -1))
    return z


# speedup vs baseline: 1.0709x; 1.0709x over previous
import jax
import jax.numpy as jnp
from jax import lax
from jax.experimental import pallas as pl
from jax.experimental.pallas import tpu as pltpu
from jax.experimental.pallas import tpu_sc as plsc

N = 10000
E = 320000
D = 512
L = 16
NC = 2
NS = 16
NT = NC * NS
WN = 320
DH = D // 2
SCAN = 3200
NCHUNK = E // SCAN
CD_SZ = SCAN + 64
LASTW = N - (NT - 1) * WN
BLK = 1000


def _edge_body(dst_hbm, src_hbm, kh_hbm, qvh_hbm, agg_hbm,
               eb_d0, eb_s0, eb_d1, eb_s1, cd_v, cs_v,
               kd0, qv0, kd1, qv1, acc, w_v,
               sem0, sem1, semA, semB):
    cid = lax.axis_index("c")
    sid = lax.axis_index("s")
    t = cid * NS + sid
    wlo = t * WN
    lane = lax.iota(jnp.int32, L)

    def issue_gather(b, n, h, kdb, qvb, sem):
        base = b * L
        d = cd_v[pl.ds(base, L)]
        s = cs_v[pl.ds(base, L)]
        valid = (lane + base) < n
        gd = jnp.where(valid, d + h * N, 0)
        gs = jnp.where(valid, s + h * N, 0)
        pltpu.async_copy(kh_hbm.at[gd], kdb, sem)
        pltpu.async_copy(qvh_hbm.at[gs], qvb, sem)

    def wait_pair(kdb, qvb, sem):
        pltpu.make_async_copy(kh_hbm.at[pl.ds(0, L)], kdb, sem).wait()
        pltpu.make_async_copy(qvh_hbm.at[pl.ds(0, L)], qvb, sem).wait()

    def consume(b, n, kdb, qvb):
        base = b * L
        d = cd_v[pl.ds(base, L)]
        valid = (lane + base) < n
        lrow = jnp.where(valid, d - wlo, 0)

        def gcol(cc, _):
            off = cc * L
            for r0 in range(L):
                kv = kdb[r0, pl.ds(off, L)]
                qv2 = qvb[r0, pl.ds(off, L)]
                vv = qvb[r0, pl.ds(off + DH, L)]
                w_v[r0, pl.ds(off, L)] = vv / (1.0 + jnp.exp(-(kv + qv2)))
            return 0
        lax.fori_loop(0, DH // L, gcol, 0)

        def col(c4, _):
            for u in range(4):
                cvec = ((c4 * 4 + u) + lane) & (DH - 1)
                wcol = plsc.load_gather(w_v, [lane, cvec])
                plsc.addupdate_scatter(acc, [lrow, cvec], wcol, mask=valid)
            return 0
        lax.fori_loop(0, DH // 4, col, 0)

    def issue_ids(c2, buf_d, buf_s, semc):
        off = c2 * SCAN
        pltpu.async_copy(dst_hbm.at[pl.ds(off, SCAN)], buf_d, semc)
        pltpu.async_copy(src_hbm.at[pl.ds(off, SCAN)], buf_s, semc)

    for h in range(2):
        def zr(r, _):
            for c in range(DH // L):
                acc[r, pl.ds(c * L, L)] = jnp.zeros((L,), jnp.float32)
            return 0
        lax.fori_loop(0, WN, zr, 0)

        def do_chunk(buf_d, buf_s, semc):
            pltpu.make_async_copy(dst_hbm.at[pl.ds(0, SCAN)], buf_d, semc).wait()
            pltpu.make_async_copy(src_hbm.at[pl.ds(0, SCAN)], buf_s, semc).wait()

            def scan_body(it, n):
                for u in range(2):
                    d = buf_d[pl.ds((it * 2 + u) * L, L)]
                    s = buf_s[pl.ds((it * 2 + u) * L, L)]
                    m = plsc.bitcast(d - wlo, jnp.uint32) < jnp.uint32(WN)
                    mi = jnp.where(m, jnp.int32(1), jnp.int32(0))
                    pos = n + plsc.cumsum(mi) - 1
                    plsc.store_scatter(cd_v, [pos], d, mask=m)
                    plsc.store_scatter(cs_v, [pos], s, mask=m)
                    n = n + jnp.sum(mi)
                return n
            n = lax.fori_loop(0, SCAN // L // 2, scan_body, jnp.int32(0))

            nb = (n + L - 1) // L

            @pl.when(nb > 0)
            def _():
                issue_gather(jnp.int32(0), n, h, kd0, qv0, sem0)

            def pipe(i, _):
                b0 = 2 * i

                @pl.when(b0 + 1 < nb)
                def _():
                    issue_gather(b0 + 1, n, h, kd1, qv1, sem1)
                wait_pair(kd0, qv0, sem0)
                consume(b0, n, kd0, qv0)

                @pl.when(b0 + 2 < nb)
                def _():
                    issue_gather(b0 + 2, n, h, kd0, qv0, sem0)

                @pl.when(b0 + 1 < nb)
                def _():
                    wait_pair(kd1, qv1, sem1)
                    consume(b0 + 1, n, kd1, qv1)
                return 0
            lax.fori_loop(0, (nb + 1) // 2, pipe, 0)

        issue_ids(jnp.int32(0), eb_d0, eb_s0, semA)

        def chunk_pair(i, _):
            issue_ids(2 * i + 1, eb_d1, eb_s1, semB)
            do_chunk(eb_d0, eb_s0, semA)

            @pl.when(2 * i + 2 < NCHUNK)
            def _():
                issue_ids(2 * i + 2, eb_d0, eb_s0, semA)
            do_chunk(eb_d1, eb_s1, semB)
            return 0
        lax.fori_loop(0, NCHUNK // 2, chunk_pair, 0)

        @pl.when(t < NT - 1)
        def _():
            pltpu.sync_copy(acc, agg_hbm.at[pl.ds(wlo, WN), pl.ds(h * DH, DH)])

        @pl.when(t == NT - 1)
        def _():
            pltpu.sync_copy(acc.at[pl.ds(0, LASTW)],
                            agg_hbm.at[pl.ds(wlo, LASTW), pl.ds(h * DH, DH)])


def _edge_agg(dst, src, kh, qvh):
    mesh = plsc.VectorSubcoreMesh(core_axis_name="c", subcore_axis_name="s",
                                  num_cores=NC, num_subcores=NS)
    f = pl.kernel(
        _edge_body,
        out_type=jax.ShapeDtypeStruct((N, D), jnp.float32),
        mesh=mesh,
        compiler_params=pltpu.CompilerParams(needs_layout_passes=False),
        scratch_types=[
            pltpu.VMEM((SCAN,), jnp.int32),
            pltpu.VMEM((SCAN,), jnp.int32),
            pltpu.VMEM((SCAN,), jnp.int32),
            pltpu.VMEM((SCAN,), jnp.int32),
            pltpu.VMEM((CD_SZ,), jnp.int32),
            pltpu.VMEM((CD_SZ,), jnp.int32),
            pltpu.VMEM((L, DH), jnp.float32),
            pltpu.VMEM((L, D), jnp.float32),
            pltpu.VMEM((L, DH), jnp.float32),
            pltpu.VMEM((L, D), jnp.float32),
            pltpu.VMEM((WN, DH), jnp.float32),
            pltpu.VMEM((L, DH), jnp.float32),
            pltpu.SemaphoreType.DMA,
            pltpu.SemaphoreType.DMA,
            pltpu.SemaphoreType.DMA,
            pltpu.SemaphoreType.DMA,
        ],
    )
    return f(dst, src, kh, qvh)


def _linstats_body(x_ref, W_ref, b_ref, o_ref, st_ref):
    i = pl.program_id(0)
    o = jnp.dot(x_ref[...], W_ref[...], preferred_element_type=jnp.float32) + b_ref[...]
    o_ref[...] = o

    @pl.when(i == 0)
    def _():
        st_ref[...] = jnp.zeros_like(st_ref)
    s1 = jnp.sum(o, axis=0)[None, :]
    s2 = jnp.sum(o * o, axis=0)[None, :]
    st_ref[...] += jnp.concatenate([s1, s2], axis=0)


def _linstats(x, W, b):
    K = x.shape[1]
    return pl.pallas_call(
        _linstats_body,
        grid=(N // BLK,),
        in_specs=[
            pl.BlockSpec((BLK, K), lambda i: (i, 0)),
            pl.BlockSpec((K, D), lambda i: (0, 0)),
            pl.BlockSpec((1, D), lambda i: (0, 0)),
        ],
        out_specs=[
            pl.BlockSpec((BLK, D), lambda i: (i, 0)),
            pl.BlockSpec((2, D), lambda i: (0, 0)),
        ],
        out_shape=[
            jax.ShapeDtypeStruct((N, D), jnp.float32),
            jax.ShapeDtypeStruct((2, D), jnp.float32),
        ],
    )(x, W, b)


def _bn_block(o, st_ref, g_ref, bt_ref):
    m = st_ref[0:1, :] / N
    v = st_ref[1:2, :] / N - m * m
    o = g_ref[...] * (o - m) * lax.rsqrt(v + 1e-5) + bt_ref[...]
    o = jnp.maximum(o, 0.0)
    nrm = jnp.sqrt(jnp.sum(o * o, axis=1, keepdims=True))
    return o / jnp.maximum(nrm, 1e-12)


def _bnact_body(o_ref, st_ref, g_ref, bt_ref, h_ref):
    h_ref[...] = _bn_block(o_ref[...], st_ref, g_ref, bt_ref)


def _bnact(o, st, g, bt):
    return pl.pallas_call(
        _bnact_body,
        grid=(N // BLK,),
        in_specs=[
            pl.BlockSpec((BLK, D), lambda i: (i, 0)),
            pl.BlockSpec((2, D), lambda i: (0, 0)),
            pl.BlockSpec((1, D), lambda i: (0, 0)),
            pl.BlockSpec((1, D), lambda i: (0, 0)),
        ],
        out_specs=pl.BlockSpec((BLK, D), lambda i: (i, 0)),
        out_shape=jax.ShapeDtypeStruct((N, D), jnp.float32),
    )(o, st, g, bt)


def _mm_body(h_ref, W1_ref, W2_ref, W34_ref, hw1_ref, kh_ref, qvh_ref):
    h = h_ref[...]
    hw1_ref[...] = jnp.dot(h, W1_ref[...], preferred_element_type=jnp.float32)
    k = jnp.dot(h, W2_ref[...], preferred_element_type=jnp.float32)
    kh_ref[0] = k[:, :DH]
    kh_ref[1] = k[:, DH:]
    qv = jnp.dot(h, W34_ref[...], preferred_element_type=jnp.float32)
    qvh_ref[0] = jnp.concatenate([qv[:, :DH], qv[:, D:D + DH]], axis=1)
    qvh_ref[1] = jnp.concatenate([qv[:, DH:D], qv[:, D + DH:]], axis=1)


def _mm(h, W1, W2, W34):
    return pl.pallas_call(
        _mm_body,
        grid=(N // BLK,),
        in_specs=[
            pl.BlockSpec((BLK, D), lambda i: (i, 0)),
            pl.BlockSpec((D, D), lambda i: (0, 0)),
            pl.BlockSpec((D, D), lambda i: (0, 0)),
            pl.BlockSpec((D, 2 * D), lambda i: (0, 0)),
        ],
        out_specs=[
            pl.BlockSpec((BLK, D), lambda i: (i, 0)),
            pl.BlockSpec((2, BLK, DH), lambda i: (0, i, 0)),
            pl.BlockSpec((2, BLK, D), lambda i: (0, i, 0)),
        ],
        out_shape=[
            jax.ShapeDtypeStruct((N, D), jnp.float32),
            jax.ShapeDtypeStruct((2, N, DH), jnp.float32),
            jax.ShapeDtypeStruct((2, N, D), jnp.float32),
        ],
    )(h, W1, W2, W34)


def _stats_body(hw1_ref, agg_ref, wb_ref, st_ref):
    i = pl.program_id(0)
    o = hw1_ref[...] + agg_ref[...] + wb_ref[...]

    @pl.when(i == 0)
    def _():
        st_ref[...] = jnp.zeros_like(st_ref)
    s1 = jnp.sum(o, axis=0)[None, :]
    s2 = jnp.sum(o * o, axis=0)[None, :]
    st_ref[...] += jnp.concatenate([s1, s2], axis=0)


def _stats(hw1, agg, wb):
    return pl.pallas_call(
        _stats_body,
        grid=(N // BLK,),
        in_specs=[
            pl.BlockSpec((BLK, D), lambda i: (i, 0)),
            pl.BlockSpec((BLK, D), lambda i: (i, 0)),
            pl.BlockSpec((1, D), lambda i: (0, 0)),
        ],
        out_specs=pl.BlockSpec((2, D), lambda i: (0, 0)),
        out_shape=jax.ShapeDtypeStruct((2, D), jnp.float32),
    )(hw1, agg, wb)


def _resupd_body(h_ref, hw1_ref, agg_ref, wb_ref, st_ref, g_ref, bt_ref, hn_ref):
    o = hw1_ref[...] + agg_ref[...] + wb_ref[...]
    o = _bn_block(o, st_ref, g_ref, bt_ref)
    hn_ref[...] = h_ref[...] + o


def _resupd(h, hw1, agg, wb, st, g, bt):
    return pl.pallas_call(
        _resupd_body,
        grid=(N // BLK,),
        in_specs=[
            pl.BlockSpec((BLK, D), lambda i: (i, 0)),
            pl.BlockSpec((BLK, D), lambda i: (i, 0)),
            pl.BlockSpec((BLK, D), lambda i: (i, 0)),
            pl.BlockSpec((1, D), lambda i: (0, 0)),
            pl.BlockSpec((2, D), lambda i: (0, 0)),
            pl.BlockSpec((1, D), lambda i: (0, 0)),
            pl.BlockSpec((1, D), lambda i: (0, 0)),
        ],
        out_specs=pl.BlockSpec((BLK, D), lambda i: (i, 0)),
        out_shape=jax.ShapeDtypeStruct((N, D), jnp.float32),
    )(h, hw1, agg, wb, st, g, bt)


def _post1_body(h_ref, W_ref, b_ref, z1_ref, st_ref):
    i = pl.program_id(0)
    h = h_ref[...]
    nrm = jnp.sqrt(jnp.sum(h * h, axis=1, keepdims=True))
    h = h / jnp.maximum(nrm, 1e-12)
    o = jnp.dot(h, W_ref[...], preferred_element_type=jnp.float32) + b_ref[...]
    z1_ref[...] = o

    @pl.when(i == 0)
    def _():
        st_ref[...] = jnp.zeros_like(st_ref)
    s1 = jnp.sum(o, axis=0)[None, :]
    s2 = jnp.sum(o * o, axis=0)[None, :]
    st_ref[...] += jnp.concatenate([s1, s2], axis=0)


def _post1(h, W, b):
    return pl.pallas_call(
        _post1_body,
        grid=(N // BLK,),
        in_specs=[
            pl.BlockSpec((BLK, D), lambda i: (i, 0)),
            pl.BlockSpec((D, D), lambda i: (0, 0)),
            pl.BlockSpec((1, D), lambda i: (0, 0)),
        ],
        out_specs=[
            pl.BlockSpec((BLK, D), lambda i: (i, 0)),
            pl.BlockSpec((2, D), lambda i: (0, 0)),
        ],
        out_shape=[
            jax.ShapeDtypeStruct((N, D), jnp.float32),
            jax.ShapeDtypeStruct((2, D), jnp.float32),
        ],
    )(h, W, b)


def _post2_body(z1_ref, st_ref, g_ref, bt_ref, W2_ref, b2_ref, z_ref):
    o = z1_ref[...]
    m = st_ref[0:1, :] / N
    v = st_ref[1:2, :] / N - m * m
    o = g_ref[...] * (o - m) * lax.rsqrt(v + 1e-5) + bt_ref[...]
    o = jnp.maximum(o, 0.0)
    z_ref[...] = jnp.dot(o, W2_ref[...], preferred_element_type=jnp.float32) + b2_ref[...]


def _post2(z1, st, g, bt, W2, b2):
    DO = W2.shape[1]
    return pl.pallas_call(
        _post2_body,
        grid=(N // BLK,),
        in_specs=[
            pl.BlockSpec((BLK, D), lambda i: (i, 0)),
            pl.BlockSpec((2, D), lambda i: (0, 0)),
            pl.BlockSpec((1, D), lambda i: (0, 0)),
            pl.BlockSpec((1, D), lambda i: (0, 0)),
            pl.BlockSpec((D, DO), lambda i: (0, 0)),
            pl.BlockSpec((1, DO), lambda i: (0, 0)),
        ],
        out_specs=pl.BlockSpec((BLK, DO), lambda i: (i, 0)),
        out_shape=jax.ShapeDtypeStruct((N, DO), jnp.float32),
    )(z1, st, g, bt, W2, b2)


def kernel(x, edge_index, pre_W, pre_b, pre_g, pre_bt, W1, W2, W3, W4, Wb,
           g_mp, b_mp, h1_W, h1_b, h1_g, h1_bt, h2_W, h2_b):
    src = edge_index[0].astype(jnp.int32)
    dst = edge_index[1].astype(jnp.int32)

    pre, st = _linstats(x, pre_W, pre_b.reshape(1, D))
    h = _bnact(pre, st, pre_g.reshape(1, D), pre_bt.reshape(1, D))

    W34 = jnp.concatenate([W3, W4], axis=-1)
    xs = (W1, W2, W34, Wb[:, None, :], g_mp[:, None, :], b_mp[:, None, :])

    def layer(h, ws):
        W1l, W2l, W34l, wbl, gl, bl = ws
        hw1, kh, qvh = _mm(h, W1l, W2l, W34l)
        agg = _edge_agg(dst, src, kh.reshape(2 * N, DH), qvh.reshape(2 * N, D))
        stl = _stats(hw1, agg, wbl)
        h = _resupd(h, hw1, agg, wbl, stl, gl, bl)
        return h, None

    h, _ = lax.scan(layer, h, xs)

    z1, st = _post1(h, h1_W, h1_b.reshape(1, D))
    z = _post2(z1, st, h1_g.reshape(1, D), h1_bt.reshape(1, D),
               h2_W, h2_b.reshape(1, -1))
    return z


# speedup vs baseline: 1.2337x; 1.1520x over previous
import jax
import jax.numpy as jnp
from jax import lax
from jax.experimental import pallas as pl
from jax.experimental.pallas import tpu as pltpu
from jax.experimental.pallas import tpu_sc as plsc

N = 10000
E = 320000
D = 512
L = 16
NC = 2
NS = 16
NT = NC * NS
WN = 320
DH = D // 2
SCAN = 3200
NCHUNK = E // SCAN
CD_SZ = SCAN + 64
LASTW = N - (NT - 1) * WN
BLK = 1000


def _edge_body(dst_hbm, src_hbm, kh_hbm, qvh_hbm, agg_hbm,
               eb_d0, eb_s0, eb_d1, eb_s1, cd_v, cs_v,
               kd0, qv0, kd1, qv1, acc, w_v,
               sem0, sem1, semA, semB):
    cid = lax.axis_index("c")
    sid = lax.axis_index("s")
    t = cid * NS + sid
    wlo = t * WN
    lane = lax.iota(jnp.int32, L)

    def issue_gather(b, n, h, kdb, qvb, sem):
        base = b * L
        d = cd_v[pl.ds(base, L)]
        s = cs_v[pl.ds(base, L)]
        valid = (lane + base) < n
        gd = jnp.where(valid, d + h * N, 0)
        gs = jnp.where(valid, s + h * N, 0)
        pltpu.async_copy(kh_hbm.at[gd], kdb, sem)
        pltpu.async_copy(qvh_hbm.at[gs], qvb, sem)

    def wait_pair(kdb, qvb, sem):
        pltpu.make_async_copy(kh_hbm.at[pl.ds(0, L)], kdb, sem).wait()
        pltpu.make_async_copy(qvh_hbm.at[pl.ds(0, L)], qvb, sem).wait()

    def consume(b, n, kdb, qvb):
        base = b * L
        d = cd_v[pl.ds(base, L)]
        valid = (lane + base) < n
        lrow = jnp.where(valid, d - wlo, 0)

        def col(c4, _):
            for u in range(4):
                cvec = ((c4 * 4 + u) + lane) & (DH - 1)
                kcol = plsc.load_gather(kdb, [lane, cvec])
                qcol = plsc.load_gather(qvb, [lane, cvec])
                vcol = plsc.load_gather(qvb, [lane, cvec + DH])
                g = vcol / (1.0 + jnp.exp(-(kcol + qcol)))
                plsc.addupdate_scatter(acc, [lrow, cvec], g, mask=valid)
            return 0
        lax.fori_loop(0, DH // 4, col, 0)

    def issue_ids(c2, buf_d, buf_s, semc):
        off = c2 * SCAN
        pltpu.async_copy(dst_hbm.at[pl.ds(off, SCAN)], buf_d, semc)
        pltpu.async_copy(src_hbm.at[pl.ds(off, SCAN)], buf_s, semc)

    for h in range(2):
        def zr(r, _):
            for c in range(DH // L):
                acc[r, pl.ds(c * L, L)] = jnp.zeros((L,), jnp.float32)
            return 0
        lax.fori_loop(0, WN, zr, 0)

        def do_chunk(buf_d, buf_s, semc):
            pltpu.make_async_copy(dst_hbm.at[pl.ds(0, SCAN)], buf_d, semc).wait()
            pltpu.make_async_copy(src_hbm.at[pl.ds(0, SCAN)], buf_s, semc).wait()

            def scan_body(it, n):
                for u in range(2):
                    d = buf_d[pl.ds((it * 2 + u) * L, L)]
                    s = buf_s[pl.ds((it * 2 + u) * L, L)]
                    m = plsc.bitcast(d - wlo, jnp.uint32) < jnp.uint32(WN)
                    mi = jnp.where(m, jnp.int32(1), jnp.int32(0))
                    pos = n + plsc.cumsum(mi) - 1
                    plsc.store_scatter(cd_v, [pos], d, mask=m)
                    plsc.store_scatter(cs_v, [pos], s, mask=m)
                    n = n + jnp.sum(mi)
                return n
            n = lax.fori_loop(0, SCAN // L // 2, scan_body, jnp.int32(0))

            nb = (n + L - 1) // L

            @pl.when(nb > 0)
            def _():
                issue_gather(jnp.int32(0), n, h, kd0, qv0, sem0)

            def pipe(i, _):
                b0 = 2 * i

                @pl.when(b0 + 1 < nb)
                def _():
                    issue_gather(b0 + 1, n, h, kd1, qv1, sem1)
                wait_pair(kd0, qv0, sem0)
                consume(b0, n, kd0, qv0)

                @pl.when(b0 + 2 < nb)
                def _():
                    issue_gather(b0 + 2, n, h, kd0, qv0, sem0)

                @pl.when(b0 + 1 < nb)
                def _():
                    wait_pair(kd1, qv1, sem1)
                    consume(b0 + 1, n, kd1, qv1)
                return 0
            lax.fori_loop(0, (nb + 1) // 2, pipe, 0)

        issue_ids(jnp.int32(0), eb_d0, eb_s0, semA)

        def chunk_pair(i, _):
            issue_ids(2 * i + 1, eb_d1, eb_s1, semB)
            do_chunk(eb_d0, eb_s0, semA)

            @pl.when(2 * i + 2 < NCHUNK)
            def _():
                issue_ids(2 * i + 2, eb_d0, eb_s0, semA)
            do_chunk(eb_d1, eb_s1, semB)
            return 0
        lax.fori_loop(0, NCHUNK // 2, chunk_pair, 0)

        @pl.when(t < NT - 1)
        def _():
            pltpu.sync_copy(acc, agg_hbm.at[pl.ds(wlo, WN), pl.ds(h * DH, DH)])

        @pl.when(t == NT - 1)
        def _():
            pltpu.sync_copy(acc.at[pl.ds(0, LASTW)],
                            agg_hbm.at[pl.ds(wlo, LASTW), pl.ds(h * DH, DH)])


def _edge_agg(dst, src, kh, qvh):
    mesh = plsc.VectorSubcoreMesh(core_axis_name="c", subcore_axis_name="s",
                                  num_cores=NC, num_subcores=NS)
    f = pl.kernel(
        _edge_body,
        out_type=jax.ShapeDtypeStruct((N, D), jnp.float32),
        mesh=mesh,
        compiler_params=pltpu.CompilerParams(needs_layout_passes=False),
        scratch_types=[
            pltpu.VMEM((SCAN,), jnp.int32),
            pltpu.VMEM((SCAN,), jnp.int32),
            pltpu.VMEM((SCAN,), jnp.int32),
            pltpu.VMEM((SCAN,), jnp.int32),
            pltpu.VMEM((CD_SZ,), jnp.int32),
            pltpu.VMEM((CD_SZ,), jnp.int32),
            pltpu.VMEM((L, DH), jnp.float32),
            pltpu.VMEM((L, D), jnp.float32),
            pltpu.VMEM((L, DH), jnp.float32),
            pltpu.VMEM((L, D), jnp.float32),
            pltpu.VMEM((WN, DH), jnp.float32),
            pltpu.VMEM((L, DH), jnp.float32),
            pltpu.SemaphoreType.DMA,
            pltpu.SemaphoreType.DMA,
            pltpu.SemaphoreType.DMA,
            pltpu.SemaphoreType.DMA,
        ],
    )
    return f(dst, src, kh, qvh)


def _linstats_body(x_ref, W_ref, b_ref, o_ref, st_ref):
    i = pl.program_id(0)
    o = jnp.dot(x_ref[...], W_ref[...], preferred_element_type=jnp.float32) + b_ref[...]
    o_ref[...] = o

    @pl.when(i == 0)
    def _():
        st_ref[...] = jnp.zeros_like(st_ref)
    s1 = jnp.sum(o, axis=0)[None, :]
    s2 = jnp.sum(o * o, axis=0)[None, :]
    st_ref[...] += jnp.concatenate([s1, s2], axis=0)


def _linstats(x, W, b):
    K = x.shape[1]
    return pl.pallas_call(
        _linstats_body,
        grid=(N // BLK,),
        in_specs=[
            pl.BlockSpec((BLK, K), lambda i: (i, 0)),
            pl.BlockSpec((K, D), lambda i: (0, 0)),
            pl.BlockSpec((1, D), lambda i: (0, 0)),
        ],
        out_specs=[
            pl.BlockSpec((BLK, D), lambda i: (i, 0)),
            pl.BlockSpec((2, D), lambda i: (0, 0)),
        ],
        out_shape=[
            jax.ShapeDtypeStruct((N, D), jnp.float32),
            jax.ShapeDtypeStruct((2, D), jnp.float32),
        ],
    )(x, W, b)


def _bn_block(o, st_ref, g_ref, bt_ref):
    m = st_ref[0:1, :] / N
    v = st_ref[1:2, :] / N - m * m
    o = g_ref[...] * (o - m) * lax.rsqrt(v + 1e-5) + bt_ref[...]
    o = jnp.maximum(o, 0.0)
    nrm = jnp.sqrt(jnp.sum(o * o, axis=1, keepdims=True))
    return o / jnp.maximum(nrm, 1e-12)


def _bnact_body(o_ref, st_ref, g_ref, bt_ref, h_ref):
    h_ref[...] = _bn_block(o_ref[...], st_ref, g_ref, bt_ref)


def _bnact(o, st, g, bt):
    return pl.pallas_call(
        _bnact_body,
        grid=(N // BLK,),
        in_specs=[
            pl.BlockSpec((BLK, D), lambda i: (i, 0)),
            pl.BlockSpec((2, D), lambda i: (0, 0)),
            pl.BlockSpec((1, D), lambda i: (0, 0)),
            pl.BlockSpec((1, D), lambda i: (0, 0)),
        ],
        out_specs=pl.BlockSpec((BLK, D), lambda i: (i, 0)),
        out_shape=jax.ShapeDtypeStruct((N, D), jnp.float32),
    )(o, st, g, bt)


def _mm_body(h_ref, W1_ref, W2_ref, W34_ref, hw1_ref, kh_ref, qvh_ref):
    h = h_ref[...]
    hw1_ref[...] = jnp.dot(h, W1_ref[...], preferred_element_type=jnp.float32)
    k = jnp.dot(h, W2_ref[...], preferred_element_type=jnp.float32)
    kh_ref[0] = k[:, :DH]
    kh_ref[1] = k[:, DH:]
    qv = jnp.dot(h, W34_ref[...], preferred_element_type=jnp.float32)
    qvh_ref[0] = jnp.concatenate([qv[:, :DH], qv[:, D:D + DH]], axis=1)
    qvh_ref[1] = jnp.concatenate([qv[:, DH:D], qv[:, D + DH:]], axis=1)


def _mm(h, W1, W2, W34):
    return pl.pallas_call(
        _mm_body,
        grid=(N // BLK,),
        in_specs=[
            pl.BlockSpec((BLK, D), lambda i: (i, 0)),
            pl.BlockSpec((D, D), lambda i: (0, 0)),
            pl.BlockSpec((D, D), lambda i: (0, 0)),
            pl.BlockSpec((D, 2 * D), lambda i: (0, 0)),
        ],
        out_specs=[
            pl.BlockSpec((BLK, D), lambda i: (i, 0)),
            pl.BlockSpec((2, BLK, DH), lambda i: (0, i, 0)),
            pl.BlockSpec((2, BLK, D), lambda i: (0, i, 0)),
        ],
        out_shape=[
            jax.ShapeDtypeStruct((N, D), jnp.float32),
            jax.ShapeDtypeStruct((2, N, DH), jnp.float32),
            jax.ShapeDtypeStruct((2, N, D), jnp.float32),
        ],
    )(h, W1, W2, W34)


def _stats_body(hw1_ref, agg_ref, wb_ref, st_ref):
    i = pl.program_id(0)
    o = hw1_ref[...] + agg_ref[...] + wb_ref[...]

    @pl.when(i == 0)
    def _():
        st_ref[...] = jnp.zeros_like(st_ref)
    s1 = jnp.sum(o, axis=0)[None, :]
    s2 = jnp.sum(o * o, axis=0)[None, :]
    st_ref[...] += jnp.concatenate([s1, s2], axis=0)


def _stats(hw1, agg, wb):
    return pl.pallas_call(
        _stats_body,
        grid=(N // BLK,),
        in_specs=[
            pl.BlockSpec((BLK, D), lambda i: (i, 0)),
            pl.BlockSpec((BLK, D), lambda i: (i, 0)),
            pl.BlockSpec((1, D), lambda i: (0, 0)),
        ],
        out_specs=pl.BlockSpec((2, D), lambda i: (0, 0)),
        out_shape=jax.ShapeDtypeStruct((2, D), jnp.float32),
    )(hw1, agg, wb)


def _resupd_body(h_ref, hw1_ref, agg_ref, wb_ref, st_ref, g_ref, bt_ref, hn_ref):
    o = hw1_ref[...] + agg_ref[...] + wb_ref[...]
    o = _bn_block(o, st_ref, g_ref, bt_ref)
    hn_ref[...] = h_ref[...] + o


def _resupd(h, hw1, agg, wb, st, g, bt):
    return pl.pallas_call(
        _resupd_body,
        grid=(N // BLK,),
        in_specs=[
            pl.BlockSpec((BLK, D), lambda i: (i, 0)),
            pl.BlockSpec((BLK, D), lambda i: (i, 0)),
            pl.BlockSpec((BLK, D), lambda i: (i, 0)),
            pl.BlockSpec((1, D), lambda i: (0, 0)),
            pl.BlockSpec((2, D), lambda i: (0, 0)),
            pl.BlockSpec((1, D), lambda i: (0, 0)),
            pl.BlockSpec((1, D), lambda i: (0, 0)),
        ],
        out_specs=pl.BlockSpec((BLK, D), lambda i: (i, 0)),
        out_shape=jax.ShapeDtypeStruct((N, D), jnp.float32),
    )(h, hw1, agg, wb, st, g, bt)


def _post1_body(h_ref, W_ref, b_ref, z1_ref, st_ref):
    i = pl.program_id(0)
    h = h_ref[...]
    nrm = jnp.sqrt(jnp.sum(h * h, axis=1, keepdims=True))
    h = h / jnp.maximum(nrm, 1e-12)
    o = jnp.dot(h, W_ref[...], preferred_element_type=jnp.float32) + b_ref[...]
    z1_ref[...] = o

    @pl.when(i == 0)
    def _():
        st_ref[...] = jnp.zeros_like(st_ref)
    s1 = jnp.sum(o, axis=0)[None, :]
    s2 = jnp.sum(o * o, axis=0)[None, :]
    st_ref[...] += jnp.concatenate([s1, s2], axis=0)


def _post1(h, W, b):
    return pl.pallas_call(
        _post1_body,
        grid=(N // BLK,),
        in_specs=[
            pl.BlockSpec((BLK, D), lambda i: (i, 0)),
            pl.BlockSpec((D, D), lambda i: (0, 0)),
            pl.BlockSpec((1, D), lambda i: (0, 0)),
        ],
        out_specs=[
            pl.BlockSpec((BLK, D), lambda i: (i, 0)),
            pl.BlockSpec((2, D), lambda i: (0, 0)),
        ],
        out_shape=[
            jax.ShapeDtypeStruct((N, D), jnp.float32),
            jax.ShapeDtypeStruct((2, D), jnp.float32),
        ],
    )(h, W, b)


def _post2_body(z1_ref, st_ref, g_ref, bt_ref, W2_ref, b2_ref, z_ref):
    o = z1_ref[...]
    m = st_ref[0:1, :] / N
    v = st_ref[1:2, :] / N - m * m
    o = g_ref[...] * (o - m) * lax.rsqrt(v + 1e-5) + bt_ref[...]
    o = jnp.maximum(o, 0.0)
    z_ref[...] = jnp.dot(o, W2_ref[...], preferred_element_type=jnp.float32) + b2_ref[...]


def _post2(z1, st, g, bt, W2, b2):
    DO = W2.shape[1]
    return pl.pallas_call(
        _post2_body,
        grid=(N // BLK,),
        in_specs=[
            pl.BlockSpec((BLK, D), lambda i: (i, 0)),
            pl.BlockSpec((2, D), lambda i: (0, 0)),
            pl.BlockSpec((1, D), lambda i: (0, 0)),
            pl.BlockSpec((1, D), lambda i: (0, 0)),
            pl.BlockSpec((D, DO), lambda i: (0, 0)),
            pl.BlockSpec((1, DO), lambda i: (0, 0)),
        ],
        out_specs=pl.BlockSpec((BLK, DO), lambda i: (i, 0)),
        out_shape=jax.ShapeDtypeStruct((N, DO), jnp.float32),
    )(z1, st, g, bt, W2, b2)


def kernel(x, edge_index, pre_W, pre_b, pre_g, pre_bt, W1, W2, W3, W4, Wb,
           g_mp, b_mp, h1_W, h1_b, h1_g, h1_bt, h2_W, h2_b):
    src = edge_index[0].astype(jnp.int32)
    dst = edge_index[1].astype(jnp.int32)

    pre, st = _linstats(x, pre_W, pre_b.reshape(1, D))
    h = _bnact(pre, st, pre_g.reshape(1, D), pre_bt.reshape(1, D))

    W34 = jnp.concatenate([W3, W4], axis=-1)
    xs = (W1, W2, W34, Wb[:, None, :], g_mp[:, None, :], b_mp[:, None, :])

    def layer(h, ws):
        W1l, W2l, W34l, wbl, gl, bl = ws
        hw1, kh, qvh = _mm(h, W1l, W2l, W34l)
        agg = _edge_agg(dst, src, kh.reshape(2 * N, DH), qvh.reshape(2 * N, D))
        stl = _stats(hw1, agg, wbl)
        h = _resupd(h, hw1, agg, wbl, stl, gl, bl)
        return h, None

    h, _ = lax.scan(layer, h, xs)

    z1, st = _post1(h, h1_W, h1_b.reshape(1, D))
    z = _post2(z1, st, h1_g.reshape(1, D), h1_bt.reshape(1, D),
               h2_W, h2_b.reshape(1, -1))
    return z


# speedup vs baseline: 3.7330x; 3.0258x over previous
import jax
import jax.numpy as jnp
from jax import lax
from jax.experimental import pallas as pl
from jax.experimental.pallas import tpu as pltpu
from jax.experimental.pallas import tpu_sc as plsc

N = 10000
E = 320000
D = 512
L = 16
NC = 2
NS = 16
NT = NC * NS
WN = 320
DH = D // 2
SCAN = 3200
NCHUNK = E // SCAN
CD_SZ = SCAN + 64
LASTW = N - (NT - 1) * WN
BLK = 1000


def _edge_body(dst_hbm, src_hbm, kh_hbm, qvh_hbm, agg_hbm,
               eb_d0, eb_s0, eb_d1, eb_s1, cd_v, cs_v,
               kd0, qv0, kd1, qv1, acc, w_v,
               sem0, sem1, semA, semB):
    cid = lax.axis_index("c")
    sid = lax.axis_index("s")
    t = cid * NS + sid
    wlo = t * WN
    lane = lax.iota(jnp.int32, L)

    def issue_gather(b, n, h, kdb, qvb, sem):
        base = b * L
        d = cd_v[pl.ds(base, L)]
        s = cs_v[pl.ds(base, L)]
        valid = (lane + base) < n
        gd = jnp.where(valid, d + h * N, 0)
        gs = jnp.where(valid, s + h * N, 0)
        pltpu.async_copy(kh_hbm.at[gd], kdb, sem)
        pltpu.async_copy(qvh_hbm.at[gs], qvb, sem)

    def wait_pair(kdb, qvb, sem):
        pltpu.make_async_copy(kh_hbm.at[pl.ds(0, L)], kdb, sem).wait()
        pltpu.make_async_copy(qvh_hbm.at[pl.ds(0, L)], qvb, sem).wait()

    def consume(b, n, kdb, qvb):
        base = b * L
        d = cd_v[pl.ds(base, L)]
        valid = (lane + base) < n
        lrow = jnp.where(valid, d - wlo, 0)

        @plsc.parallel_loop(0, DH, unroll=8)
        def col(c):
            cvec = (c + lane) & (DH - 1)
            kcol = plsc.load_gather(kdb, [lane, cvec])
            qcol = plsc.load_gather(qvb, [lane, cvec])
            vcol = plsc.load_gather(qvb, [lane, cvec + DH])
            g = vcol / (1.0 + jnp.exp(-(kcol + qcol)))
            plsc.addupdate_scatter(acc, [lrow, cvec], g, mask=valid)

    def issue_ids(c2, buf_d, buf_s, semc):
        off = c2 * SCAN
        pltpu.async_copy(dst_hbm.at[pl.ds(off, SCAN)], buf_d, semc)
        pltpu.async_copy(src_hbm.at[pl.ds(off, SCAN)], buf_s, semc)

    for h in range(2):
        def zr(r, _):
            for c in range(DH // L):
                acc[r, pl.ds(c * L, L)] = jnp.zeros((L,), jnp.float32)
            return 0
        lax.fori_loop(0, WN, zr, 0)

        def do_chunk(buf_d, buf_s, semc):
            pltpu.make_async_copy(dst_hbm.at[pl.ds(0, SCAN)], buf_d, semc).wait()
            pltpu.make_async_copy(src_hbm.at[pl.ds(0, SCAN)], buf_s, semc).wait()

            @plsc.parallel_loop(0, SCAN // L, unroll=4, carry=jnp.int32(0))
            def scan_loop(it, n):
                d = buf_d[pl.ds(it * L, L)]
                s = buf_s[pl.ds(it * L, L)]
                m = plsc.bitcast(d - wlo, jnp.uint32) < jnp.uint32(WN)
                mi = jnp.where(m, jnp.int32(1), jnp.int32(0))
                pos = n + plsc.cumsum(mi) - 1
                plsc.store_scatter(cd_v, [pos], d, mask=m)
                plsc.store_scatter(cs_v, [pos], s, mask=m)
                return n + jnp.sum(mi)
            n = scan_loop

            nb = (n + L - 1) // L

            @pl.when(nb > 0)
            def _():
                issue_gather(jnp.int32(0), n, h, kd0, qv0, sem0)

            def pipe(i, _):
                b0 = 2 * i

                @pl.when(b0 + 1 < nb)
                def _():
                    issue_gather(b0 + 1, n, h, kd1, qv1, sem1)
                wait_pair(kd0, qv0, sem0)
                consume(b0, n, kd0, qv0)

                @pl.when(b0 + 2 < nb)
                def _():
                    issue_gather(b0 + 2, n, h, kd0, qv0, sem0)

                @pl.when(b0 + 1 < nb)
                def _():
                    wait_pair(kd1, qv1, sem1)
                    consume(b0 + 1, n, kd1, qv1)
                return 0
            lax.fori_loop(0, (nb + 1) // 2, pipe, 0)

        issue_ids(jnp.int32(0), eb_d0, eb_s0, semA)

        def chunk_pair(i, _):
            issue_ids(2 * i + 1, eb_d1, eb_s1, semB)
            do_chunk(eb_d0, eb_s0, semA)

            @pl.when(2 * i + 2 < NCHUNK)
            def _():
                issue_ids(2 * i + 2, eb_d0, eb_s0, semA)
            do_chunk(eb_d1, eb_s1, semB)
            return 0
        lax.fori_loop(0, NCHUNK // 2, chunk_pair, 0)

        @pl.when(t < NT - 1)
        def _():
            pltpu.sync_copy(acc, agg_hbm.at[pl.ds(wlo, WN), pl.ds(h * DH, DH)])

        @pl.when(t == NT - 1)
        def _():
            pltpu.sync_copy(acc.at[pl.ds(0, LASTW)],
                            agg_hbm.at[pl.ds(wlo, LASTW), pl.ds(h * DH, DH)])


def _edge_agg(dst, src, kh, qvh):
    mesh = plsc.VectorSubcoreMesh(core_axis_name="c", subcore_axis_name="s",
                                  num_cores=NC, num_subcores=NS)
    f = pl.kernel(
        _edge_body,
        out_type=jax.ShapeDtypeStruct((N, D), jnp.float32),
        mesh=mesh,
        compiler_params=pltpu.CompilerParams(needs_layout_passes=False),
        scratch_types=[
            pltpu.VMEM((SCAN,), jnp.int32),
            pltpu.VMEM((SCAN,), jnp.int32),
            pltpu.VMEM((SCAN,), jnp.int32),
            pltpu.VMEM((SCAN,), jnp.int32),
            pltpu.VMEM((CD_SZ,), jnp.int32),
            pltpu.VMEM((CD_SZ,), jnp.int32),
            pltpu.VMEM((L, DH), jnp.float32),
            pltpu.VMEM((L, D), jnp.float32),
            pltpu.VMEM((L, DH), jnp.float32),
            pltpu.VMEM((L, D), jnp.float32),
            pltpu.VMEM((WN, DH), jnp.float32),
            pltpu.VMEM((L, DH), jnp.float32),
            pltpu.SemaphoreType.DMA,
            pltpu.SemaphoreType.DMA,
            pltpu.SemaphoreType.DMA,
            pltpu.SemaphoreType.DMA,
        ],
    )
    return f(dst, src, kh, qvh)


def _linstats_body(x_ref, W_ref, b_ref, o_ref, st_ref):
    i = pl.program_id(0)
    o = jnp.dot(x_ref[...], W_ref[...], preferred_element_type=jnp.float32) + b_ref[...]
    o_ref[...] = o

    @pl.when(i == 0)
    def _():
        st_ref[...] = jnp.zeros_like(st_ref)
    s1 = jnp.sum(o, axis=0)[None, :]
    s2 = jnp.sum(o * o, axis=0)[None, :]
    st_ref[...] += jnp.concatenate([s1, s2], axis=0)


def _linstats(x, W, b):
    K = x.shape[1]
    return pl.pallas_call(
        _linstats_body,
        grid=(N // BLK,),
        in_specs=[
            pl.BlockSpec((BLK, K), lambda i: (i, 0)),
            pl.BlockSpec((K, D), lambda i: (0, 0)),
            pl.BlockSpec((1, D), lambda i: (0, 0)),
        ],
        out_specs=[
            pl.BlockSpec((BLK, D), lambda i: (i, 0)),
            pl.BlockSpec((2, D), lambda i: (0, 0)),
        ],
        out_shape=[
            jax.ShapeDtypeStruct((N, D), jnp.float32),
            jax.ShapeDtypeStruct((2, D), jnp.float32),
        ],
    )(x, W, b)


def _bn_block(o, st_ref, g_ref, bt_ref):
    m = st_ref[0:1, :] / N
    v = st_ref[1:2, :] / N - m * m
    o = g_ref[...] * (o - m) * lax.rsqrt(v + 1e-5) + bt_ref[...]
    o = jnp.maximum(o, 0.0)
    nrm = jnp.sqrt(jnp.sum(o * o, axis=1, keepdims=True))
    return o / jnp.maximum(nrm, 1e-12)


def _bnact_body(o_ref, st_ref, g_ref, bt_ref, h_ref):
    h_ref[...] = _bn_block(o_ref[...], st_ref, g_ref, bt_ref)


def _bnact(o, st, g, bt):
    return pl.pallas_call(
        _bnact_body,
        grid=(N // BLK,),
        in_specs=[
            pl.BlockSpec((BLK, D), lambda i: (i, 0)),
            pl.BlockSpec((2, D), lambda i: (0, 0)),
            pl.BlockSpec((1, D), lambda i: (0, 0)),
            pl.BlockSpec((1, D), lambda i: (0, 0)),
        ],
        out_specs=pl.BlockSpec((BLK, D), lambda i: (i, 0)),
        out_shape=jax.ShapeDtypeStruct((N, D), jnp.float32),
    )(o, st, g, bt)


def _mm_body(h_ref, W1_ref, W2_ref, W34_ref, hw1_ref, kh_ref, qvh_ref):
    h = h_ref[...]
    hw1_ref[...] = jnp.dot(h, W1_ref[...], preferred_element_type=jnp.float32)
    k = jnp.dot(h, W2_ref[...], preferred_element_type=jnp.float32)
    kh_ref[0] = k[:, :DH]
    kh_ref[1] = k[:, DH:]
    qv = jnp.dot(h, W34_ref[...], preferred_element_type=jnp.float32)
    qvh_ref[0] = jnp.concatenate([qv[:, :DH], qv[:, D:D + DH]], axis=1)
    qvh_ref[1] = jnp.concatenate([qv[:, DH:D], qv[:, D + DH:]], axis=1)


def _mm(h, W1, W2, W34):
    return pl.pallas_call(
        _mm_body,
        grid=(N // BLK,),
        in_specs=[
            pl.BlockSpec((BLK, D), lambda i: (i, 0)),
            pl.BlockSpec((D, D), lambda i: (0, 0)),
            pl.BlockSpec((D, D), lambda i: (0, 0)),
            pl.BlockSpec((D, 2 * D), lambda i: (0, 0)),
        ],
        out_specs=[
            pl.BlockSpec((BLK, D), lambda i: (i, 0)),
            pl.BlockSpec((2, BLK, DH), lambda i: (0, i, 0)),
            pl.BlockSpec((2, BLK, D), lambda i: (0, i, 0)),
        ],
        out_shape=[
            jax.ShapeDtypeStruct((N, D), jnp.float32),
            jax.ShapeDtypeStruct((2, N, DH), jnp.float32),
            jax.ShapeDtypeStruct((2, N, D), jnp.float32),
        ],
    )(h, W1, W2, W34)


def _stats_body(hw1_ref, agg_ref, wb_ref, st_ref):
    i = pl.program_id(0)
    o = hw1_ref[...] + agg_ref[...] + wb_ref[...]

    @pl.when(i == 0)
    def _():
        st_ref[...] = jnp.zeros_like(st_ref)
    s1 = jnp.sum(o, axis=0)[None, :]
    s2 = jnp.sum(o * o, axis=0)[None, :]
    st_ref[...] += jnp.concatenate([s1, s2], axis=0)


def _stats(hw1, agg, wb):
    return pl.pallas_call(
        _stats_body,
        grid=(N // BLK,),
        in_specs=[
            pl.BlockSpec((BLK, D), lambda i: (i, 0)),
            pl.BlockSpec((BLK, D), lambda i: (i, 0)),
            pl.BlockSpec((1, D), lambda i: (0, 0)),
        ],
        out_specs=pl.BlockSpec((2, D), lambda i: (0, 0)),
        out_shape=jax.ShapeDtypeStruct((2, D), jnp.float32),
    )(hw1, agg, wb)


def _resupd_body(h_ref, hw1_ref, agg_ref, wb_ref, st_ref, g_ref, bt_ref, hn_ref):
    o = hw1_ref[...] + agg_ref[...] + wb_ref[...]
    o = _bn_block(o, st_ref, g_ref, bt_ref)
    hn_ref[...] = h_ref[...] + o


def _resupd(h, hw1, agg, wb, st, g, bt):
    return pl.pallas_call(
        _resupd_body,
        grid=(N // BLK,),
        in_specs=[
            pl.BlockSpec((BLK, D), lambda i: (i, 0)),
            pl.BlockSpec((BLK, D), lambda i: (i, 0)),
            pl.BlockSpec((BLK, D), lambda i: (i, 0)),
            pl.BlockSpec((1, D), lambda i: (0, 0)),
            pl.BlockSpec((2, D), lambda i: (0, 0)),
            pl.BlockSpec((1, D), lambda i: (0, 0)),
            pl.BlockSpec((1, D), lambda i: (0, 0)),
        ],
        out_specs=pl.BlockSpec((BLK, D), lambda i: (i, 0)),
        out_shape=jax.ShapeDtypeStruct((N, D), jnp.float32),
    )(h, hw1, agg, wb, st, g, bt)


def _post1_body(h_ref, W_ref, b_ref, z1_ref, st_ref):
    i = pl.program_id(0)
    h = h_ref[...]
    nrm = jnp.sqrt(jnp.sum(h * h, axis=1, keepdims=True))
    h = h / jnp.maximum(nrm, 1e-12)
    o = jnp.dot(h, W_ref[...], preferred_element_type=jnp.float32) + b_ref[...]
    z1_ref[...] = o

    @pl.when(i == 0)
    def _():
        st_ref[...] = jnp.zeros_like(st_ref)
    s1 = jnp.sum(o, axis=0)[None, :]
    s2 = jnp.sum(o * o, axis=0)[None, :]
    st_ref[...] += jnp.concatenate([s1, s2], axis=0)


def _post1(h, W, b):
    return pl.pallas_call(
        _post1_body,
        grid=(N // BLK,),
        in_specs=[
            pl.BlockSpec((BLK, D), lambda i: (i, 0)),
            pl.BlockSpec((D, D), lambda i: (0, 0)),
            pl.BlockSpec((1, D), lambda i: (0, 0)),
        ],
        out_specs=[
            pl.BlockSpec((BLK, D), lambda i: (i, 0)),
            pl.BlockSpec((2, D), lambda i: (0, 0)),
        ],
        out_shape=[
            jax.ShapeDtypeStruct((N, D), jnp.float32),
            jax.ShapeDtypeStruct((2, D), jnp.float32),
        ],
    )(h, W, b)


def _post2_body(z1_ref, st_ref, g_ref, bt_ref, W2_ref, b2_ref, z_ref):
    o = z1_ref[...]
    m = st_ref[0:1, :] / N
    v = st_ref[1:2, :] / N - m * m
    o = g_ref[...] * (o - m) * lax.rsqrt(v + 1e-5) + bt_ref[...]
    o = jnp.maximum(o, 0.0)
    z_ref[...] = jnp.dot(o, W2_ref[...], preferred_element_type=jnp.float32) + b2_ref[...]


def _post2(z1, st, g, bt, W2, b2):
    DO = W2.shape[1]
    return pl.pallas_call(
        _post2_body,
        grid=(N // BLK,),
        in_specs=[
            pl.BlockSpec((BLK, D), lambda i: (i, 0)),
            pl.BlockSpec((2, D), lambda i: (0, 0)),
            pl.BlockSpec((1, D), lambda i: (0, 0)),
            pl.BlockSpec((1, D), lambda i: (0, 0)),
            pl.BlockSpec((D, DO), lambda i: (0, 0)),
            pl.BlockSpec((1, DO), lambda i: (0, 0)),
        ],
        out_specs=pl.BlockSpec((BLK, DO), lambda i: (i, 0)),
        out_shape=jax.ShapeDtypeStruct((N, DO), jnp.float32),
    )(z1, st, g, bt, W2, b2)


def kernel(x, edge_index, pre_W, pre_b, pre_g, pre_bt, W1, W2, W3, W4, Wb,
           g_mp, b_mp, h1_W, h1_b, h1_g, h1_bt, h2_W, h2_b):
    src = edge_index[0].astype(jnp.int32)
    dst = edge_index[1].astype(jnp.int32)

    pre, st = _linstats(x, pre_W, pre_b.reshape(1, D))
    h = _bnact(pre, st, pre_g.reshape(1, D), pre_bt.reshape(1, D))

    W34 = jnp.concatenate([W3, W4], axis=-1)
    xs = (W1, W2, W34, Wb[:, None, :], g_mp[:, None, :], b_mp[:, None, :])

    def layer(h, ws):
        W1l, W2l, W34l, wbl, gl, bl = ws
        hw1, kh, qvh = _mm(h, W1l, W2l, W34l)
        agg = _edge_agg(dst, src, kh.reshape(2 * N, DH), qvh.reshape(2 * N, D))
        stl = _stats(hw1, agg, wbl)
        h = _resupd(h, hw1, agg, wbl, stl, gl, bl)
        return h, None

    h, _ = lax.scan(layer, h, xs)

    z1, st = _post1(h, h1_W, h1_b.reshape(1, D))
    z = _post2(z1, st, h1_g.reshape(1, D), h1_bt.reshape(1, D),
               h2_W, h2_b.reshape(1, -1))
    return z
